# baseline (speedup 1.0000x reference)
import functools

import jax
import jax.numpy as jnp
from jax import lax
from jax.experimental import pallas as pl
from jax.experimental.pallas import tpu as pltpu
from jax.experimental.pallas import tpu_sc as plsc

_NC = 2
_NS = 16
_NW = _NC * _NS
_L = 128
_SL = 8
_NB = 4


@functools.lru_cache(maxsize=None)
def _build(V, D, NI, NJ):
    nj_hi = NJ // _SL
    ni_hi = NI // _L
    n_blocks = nj_hi * ni_hi * _SL
    assert n_blocks % _NW == 0
    bpw = n_blocks // _NW
    assert bpw % _NB == 0
    d_hi = D // _SL

    mesh = plsc.VectorSubcoreMesh(core_axis_name="c", subcore_axis_name="s")

    @functools.partial(
        pl.kernel,
        mesh=mesh,
        out_type=jax.ShapeDtypeStruct((NJ, d_hi, ni_hi, _SL, _L), jnp.float32),
        compiler_params=pltpu.CompilerParams(
            use_tc_tiling_on_sc=False, needs_layout_passes=False
        ),
        scratch_types=(
            [pltpu.VMEM((_L,), jnp.int32) for _ in range(_NB)]
            + [pltpu.VMEM((_L, D), jnp.float32) for _ in range(_NB)]
            + [pltpu.VMEM((D, _L + 1), jnp.float32) for _ in range(_NB)]
            + [pltpu.SemaphoreType.DMA for _ in range(3 * _NB)]
        ),
    )
    def gather_kernel(idx_hbm, table_hbm, out_hbm, *bufs):
        idx = bufs[0:_NB]
        rows = bufs[_NB:2 * _NB]
        tout = bufs[2 * _NB:3 * _NB]
        isem = bufs[3 * _NB:4 * _NB]
        gsem = bufs[4 * _NB:5 * _NB]
        osem = bufs[5 * _NB:6 * _NB]

        wid = lax.axis_index("s") * _NC + lax.axis_index("c")
        bid0 = wid * bpw

        iota = lax.iota(jnp.int32, 16)
        d_ivs = [iota + 16 * h for h in range(D // 16)]

        def iload(m, c):
            return pltpu.make_async_copy(idx_hbm.at[bid0 + m], idx[c], isem[c])

        def gcopy(c):
            return pltpu.make_async_copy(table_hbm.at[idx[c]], rows[c], gsem[c])

        def coords(n):
            bid = bid0 + n
            jh = bid // (ni_hi * _SL)
            ih = (bid // _SL) % ni_hi
            jl = bid % _SL
            return jh * _SL + jl, ih

        def ocopies(n, c):
            j, ib = coords(n)
            return [
                pltpu.make_async_copy(
                    tout[c].at[pl.ds(dh * _SL, _SL), pl.ds(0, _L)],
                    out_hbm.at[j, dh, ib],
                    osem[c],
                )
                for dh in range(d_hi)
            ]

        def transpose(c):
            for il in range(_L):
                col_iv = jnp.full((16,), il, jnp.int32)
                for h in range(D // 16):
                    v = rows[c][il, pl.ds(16 * h, 16)]
                    plsc.store_scatter(tout[c], [d_ivs[h], col_iv], v)

        def step(n, p):
            c = p % _NB
            c2 = (p + 2) % _NB
            gcopy(c).wait()

            @pl.when(n + _NB < bpw)
            def _():
                iload(n + _NB, c).start()

            @pl.when(n + 2 < bpw)
            def _():
                iload(n + 2, c2).wait()
                gcopy(c2).start()

            @pl.when(n >= _NB)
            def _():
                for cp in ocopies(n - _NB, c):
                    cp.wait()

            transpose(c)
            for cp in ocopies(n, c):
                cp.start()

        for m in range(_NB):
            iload(m, m).start()
        iload(0, 0).wait()
        gcopy(0).start()
        iload(1, 1).wait()
        gcopy(1).start()

        def body(t, carry):
            for p in range(_NB):
                step(_NB * t + p, p)
            return carry

        lax.fori_loop(0, bpw // _NB, body, 0)

        for m in range(_NB):
            for cp in ocopies(bpw - _NB + m, m):
                cp.wait()

    return gather_kernel


def kernel(x, pe):
    V, D = pe.shape
    NI, NJ = x.shape
    xp = (
        x.astype(jnp.int32)
        .T.reshape(NJ // _SL, _SL, NI // _L, _L)
        .transpose(0, 2, 1, 3)
        .reshape(NJ // _SL * NI // _L * _SL, _L)
    )
    k = _build(V, D, NI, NJ)(xp, pe)
    return k.transpose(2, 4, 0, 1, 3).reshape(NI, NJ, D)


# speedup vs baseline: 1.2492x; 1.2492x over previous
import functools

import jax
import jax.numpy as jnp
from jax import lax
from jax.experimental import pallas as pl
from jax.experimental.pallas import tpu as pltpu
from jax.experimental.pallas import tpu_sc as plsc

_NC = 2
_NS = 16
_NW = _NC * _NS
_L = 128
_SL = 8


@functools.lru_cache(maxsize=None)
def _build(V, D, NI, NJ):
    nj_hi = NJ // _SL
    ni_hi = NI // _L
    n_blocks = nj_hi * ni_hi * _SL
    assert n_blocks % _NW == 0
    bpw = n_blocks // _NW
    d_hi = D // _SL

    mesh = plsc.VectorSubcoreMesh(core_axis_name="c", subcore_axis_name="s")

    @functools.partial(
        pl.kernel,
        mesh=mesh,
        out_type=jax.ShapeDtypeStruct((NJ, d_hi, ni_hi, _SL, _L), jnp.float32),
        compiler_params=pltpu.CompilerParams(
            use_tc_tiling_on_sc=False, needs_layout_passes=False
        ),
        scratch_types=[
            pltpu.VMEM((_L,), jnp.int32),
            pltpu.VMEM((_L,), jnp.int32),
            pltpu.VMEM((_L, D), jnp.float32),
            pltpu.VMEM((_L, D), jnp.float32),
            pltpu.VMEM((D, _L + 1), jnp.float32),
            pltpu.VMEM((D, _L + 1), jnp.float32),
            pltpu.SemaphoreType.DMA,
            pltpu.SemaphoreType.DMA,
            pltpu.SemaphoreType.DMA,
            pltpu.SemaphoreType.DMA,
            pltpu.SemaphoreType.DMA,
            pltpu.SemaphoreType.DMA,
        ],
    )
    def gather_kernel(idx_hbm, table_hbm, out_hbm,
                      idx0, idx1, rows0, rows1, tout0, tout1,
                      isem0, isem1, gsem0, gsem1, osem0, osem1):
        wid = lax.axis_index("s") * _NC + lax.axis_index("c")
        bid0 = wid * bpw

        iota = lax.iota(jnp.int32, 16)
        d_ivs = [iota + 16 * h for h in range(D // 16)]

        def iload(m, idx_v, isem):
            return pltpu.make_async_copy(idx_hbm.at[bid0 + m], idx_v, isem)

        def gcopy(idx_v, rows_v, gsem):
            return pltpu.make_async_copy(table_hbm.at[idx_v], rows_v, gsem)

        def coords(n):
            bid = bid0 + n
            jh = bid // (ni_hi * _SL)
            ih = (bid // _SL) % ni_hi
            jl = bid % _SL
            return jh * _SL + jl, ih

        def ostart(n, tout_v, osem):
            j, ib = coords(n)
            for dh in range(d_hi):
                pltpu.make_async_copy(
                    tout_v.at[pl.ds(dh * _SL, _SL), pl.ds(0, _L)],
                    out_hbm.at[j, dh, ib],
                    osem,
                ).start()

        def owait(n, tout_v, osem):
            j, ib = coords(n)
            for dh in range(d_hi):
                pltpu.make_async_copy(
                    tout_v.at[pl.ds(dh * _SL, _SL), pl.ds(0, _L)],
                    out_hbm.at[j, dh, ib],
                    osem,
                ).wait()

        def transpose(rows_v, tout_v):
            for il0 in range(0, _L, 8):
                vs = []
                for k in range(8):
                    il = il0 + k
                    for h in range(D // 16):
                        vs.append((il, h, rows_v[il, pl.ds(16 * h, 16)]))
                for il, h, v in vs:
                    col_iv = jnp.full((16,), il, jnp.int32)
                    plsc.store_scatter(tout_v, [d_ivs[h], col_iv], v)

        def step(n, idx_c, isem_c, rows_c, gsem_c, tout_c, osem_c,
                 idx_n, isem_n, rows_n, gsem_n):
            @pl.when(n + 1 < bpw)
            def _():
                iload(n + 1, idx_n, isem_n).wait()
            gcopy(idx_c, rows_c, gsem_c).wait()
            @pl.when(n + 2 < bpw)
            def _():
                iload(n + 2, idx_c, isem_c).start()
            @pl.when(n + 1 < bpw)
            def _():
                gcopy(idx_n, rows_n, gsem_n).start()
            @pl.when(n >= 2)
            def _():
                owait(n - 2, tout_c, osem_c)
            transpose(rows_c, tout_c)
            ostart(n, tout_c, osem_c)

        iload(0, idx0, isem0).start()
        iload(1, idx1, isem1).start()
        iload(0, idx0, isem0).wait()
        gcopy(idx0, rows0, gsem0).start()

        def body(t, carry):
            n = 2 * t
            step(n, idx0, isem0, rows0, gsem0, tout0, osem0,
                 idx1, isem1, rows1, gsem1)
            step(n + 1, idx1, isem1, rows1, gsem1, tout1, osem1,
                 idx0, isem0, rows0, gsem0)
            return carry

        lax.fori_loop(0, bpw // 2, body, 0)

        owait(bpw - 2, tout0, osem0)
        owait(bpw - 1, tout1, osem1)

    return gather_kernel


def kernel(x, pe):
    V, D = pe.shape
    NI, NJ = x.shape
    xp = (
        x.astype(jnp.int32)
        .T.reshape(NJ // _SL, _SL, NI // _L, _L)
        .transpose(0, 2, 1, 3)
        .reshape(NJ // _SL * NI // _L * _SL, _L)
    )
    k = _build(V, D, NI, NJ)(xp, pe)
    return k.transpose(2, 4, 0, 1, 3).reshape(NI, NJ, D)


# speedup vs baseline: 1.3524x; 1.0827x over previous
import functools

import jax
import jax.numpy as jnp
from jax import lax
from jax.experimental import pallas as pl
from jax.experimental.pallas import tpu as pltpu
from jax.experimental.pallas import tpu_sc as plsc

_NC = 2
_NS = 16
_NW = _NC * _NS
_L = 128
_SL = 8


@functools.lru_cache(maxsize=None)
def _build(V, D, NI, NJ):
    nj_hi = NJ // _SL
    ni_hi = NI // _L
    n_blocks = nj_hi * ni_hi * _SL
    assert n_blocks % _NW == 0
    bpw = n_blocks // _NW
    d_hi = D // _SL

    mesh = plsc.VectorSubcoreMesh(core_axis_name="c", subcore_axis_name="s")

    @functools.partial(
        pl.kernel,
        mesh=mesh,
        out_type=jax.ShapeDtypeStruct((NJ, d_hi, ni_hi, _SL, _L), jnp.float32),
        compiler_params=pltpu.CompilerParams(
            use_tc_tiling_on_sc=False, needs_layout_passes=False
        ),
        scratch_types=[
            pltpu.VMEM((_L,), jnp.int32),
            pltpu.VMEM((_L,), jnp.int32),
            pltpu.VMEM((_L, D), jnp.float32),
            pltpu.VMEM((_L, D), jnp.float32),
            pltpu.VMEM((D, _L + 1), jnp.float32),
            pltpu.VMEM((D, _L + 1), jnp.float32),
            pltpu.VMEM((_L, 16), jnp.int32),
            pltpu.SemaphoreType.DMA,
            pltpu.SemaphoreType.DMA,
            pltpu.SemaphoreType.DMA,
            pltpu.SemaphoreType.DMA,
            pltpu.SemaphoreType.DMA,
            pltpu.SemaphoreType.DMA,
        ],
    )
    def gather_kernel(idx_hbm, table_hbm, out_hbm,
                      idx0, idx1, rows0, rows1, tout0, tout1, ctab,
                      isem0, isem1, gsem0, gsem1, osem0, osem1):
        wid = lax.axis_index("s") * _NC + lax.axis_index("c")
        bid0 = wid * bpw

        iota = lax.iota(jnp.int32, 16)
        d_ivs = [iota + 16 * h for h in range(D // 16)]

        one = jnp.full((16,), 1, jnp.int32)
        acc = jnp.zeros((16,), jnp.int32)
        for il in range(_L):
            ctab[il, pl.ds(0, 16)] = acc
            acc = acc + one

        def iload(m, idx_v, isem):
            return pltpu.make_async_copy(idx_hbm.at[bid0 + m], idx_v, isem)

        def gcopy(idx_v, rows_v, gsem):
            return pltpu.make_async_copy(table_hbm.at[idx_v], rows_v, gsem)

        def coords(n):
            bid = bid0 + n
            jh = bid // (ni_hi * _SL)
            ih = (bid // _SL) % ni_hi
            jl = bid % _SL
            return jh * _SL + jl, ih

        def ostart(n, tout_v, osem):
            j, ib = coords(n)
            for dh in range(d_hi):
                pltpu.make_async_copy(
                    tout_v.at[pl.ds(dh * _SL, _SL), pl.ds(0, _L)],
                    out_hbm.at[j, dh, ib],
                    osem,
                ).start()

        def owait(n, tout_v, osem):
            j, ib = coords(n)
            for dh in range(d_hi):
                pltpu.make_async_copy(
                    tout_v.at[pl.ds(dh * _SL, _SL), pl.ds(0, _L)],
                    out_hbm.at[j, dh, ib],
                    osem,
                ).wait()

        def transpose(rows_v, tout_v):
            for il0 in range(0, _L, 8):
                vs = []
                cols = {}
                for k in range(8):
                    il = il0 + k
                    cols[il] = ctab[il, pl.ds(0, 16)]
                    for h in range(D // 16):
                        vs.append((il, h, rows_v[il, pl.ds(16 * h, 16)]))
                for il, h, v in vs:
                    plsc.store_scatter(tout_v, [d_ivs[h], cols[il]], v)

        def step(n, idx_c, isem_c, rows_c, gsem_c, tout_c, osem_c,
                 idx_n, isem_n, rows_n, gsem_n):
            @pl.when(n + 1 < bpw)
            def _():
                iload(n + 1, idx_n, isem_n).wait()
            gcopy(idx_c, rows_c, gsem_c).wait()
            @pl.when(n + 2 < bpw)
            def _():
                iload(n + 2, idx_c, isem_c).start()
            @pl.when(n + 1 < bpw)
            def _():
                gcopy(idx_n, rows_n, gsem_n).start()
            @pl.when(n >= 2)
            def _():
                owait(n - 2, tout_c, osem_c)
            transpose(rows_c, tout_c)
            ostart(n, tout_c, osem_c)

        iload(0, idx0, isem0).start()
        iload(1, idx1, isem1).start()
        iload(0, idx0, isem0).wait()
        gcopy(idx0, rows0, gsem0).start()

        def body(t, carry):
            n = 2 * t
            step(n, idx0, isem0, rows0, gsem0, tout0, osem0,
                 idx1, isem1, rows1, gsem1)
            step(n + 1, idx1, isem1, rows1, gsem1, tout1, osem1,
                 idx0, isem0, rows0, gsem0)
            return carry

        lax.fori_loop(0, bpw // 2, body, 0)

        owait(bpw - 2, tout0, osem0)
        owait(bpw - 1, tout1, osem1)

    return gather_kernel


def kernel(x, pe):
    V, D = pe.shape
    NI, NJ = x.shape
    xp = (
        x.astype(jnp.int32)
        .T.reshape(NJ // _SL, _SL, NI // _L, _L)
        .transpose(0, 2, 1, 3)
        .reshape(NJ // _SL * NI // _L * _SL, _L)
    )
    k = _build(V, D, NI, NJ)(xp, pe)
    return k.transpose(2, 4, 0, 1, 3).reshape(NI, NJ, D)
